# baseline (speedup 1.0000x reference)
import jax
import jax.numpy as jnp
from jax.experimental import pallas as pl

_BM = 256


def _s1_body(uv_ref, i0_ref, a_ref):
    uv = uv_ref[...].astype(jnp.bfloat16)
    i0b = i0_ref[...].astype(jnp.bfloat16)
    acc = jax.lax.dot_general(uv, i0b, (((1,), (0,)), ((), ())),
                              preferred_element_type=jnp.float32)
    a_ref[...] = acc.astype(jnp.bfloat16)


def _s2_body(vu_ref, u0f_ref, u0sf_ref, ab_ref, i0_ref, bb_ref, spec_i_ref):
    vu = vu_ref[...].astype(jnp.bfloat16)
    dims = (((1,), (0,)), ((), ()))
    b = jax.lax.dot_general(vu, u0f_ref[...].astype(jnp.bfloat16), dims,
                            preferred_element_type=jnp.float32)
    bs = jax.lax.dot_general(vu, u0sf_ref[...].astype(jnp.bfloat16), dims,
                             preferred_element_type=jnp.float32)
    d = jax.lax.dot_general(vu, ab_ref[...], dims,
                            preferred_element_type=jnp.float32)
    f = b.shape[1]
    bb_ref[:, :f] = b.astype(jnp.bfloat16)
    bb_ref[:, f:] = bs.astype(jnp.bfloat16)
    spec_i_ref[...] = (i0_ref[...] + b + d) * (1.0 / 3.0)


def _s3_body(uv_ref, bb_ref, u0_ref, u0s_ref, ab_ref, spec_u_ref,
             share_u_ref):
    uv = uv_ref[...].astype(jnp.bfloat16)
    acc = jax.lax.dot_general(uv, bb_ref[...], (((1,), (0,)), ((), ())),
                              preferred_element_type=jnp.float32)
    f = u0_ref.shape[1]
    a = ab_ref[...].astype(jnp.float32)
    spec_u_ref[...] = (u0_ref[...] + a + acc[:, :f]) * (1.0 / 3.0)
    share_u_ref[...] = (u0s_ref[...] + a + acc[:, f:]) * (1.0 / 3.0)


def _row_spec(bm, w):
    return pl.BlockSpec((bm, w), lambda i: (i, 0))


def _full_spec(k, w):
    return pl.BlockSpec((k, w), lambda i: (0, 0))


def _domain(UV, VU, u0, u0_share, i0):
    n_u, f = u0.shape
    n_i = i0.shape[0]
    bm = _BM
    bf = jnp.bfloat16

    a_bf = pl.pallas_call(
        _s1_body,
        grid=(n_u // bm,),
        in_specs=[_row_spec(bm, n_i), _full_spec(n_i, f)],
        out_specs=_row_spec(bm, f),
        out_shape=jax.ShapeDtypeStruct((n_u, f), bf),
    )(UV, i0)

    bb_bf, spec_i = pl.pallas_call(
        _s2_body,
        grid=(n_i // bm,),
        in_specs=[_row_spec(bm, n_u), _full_spec(n_u, f), _full_spec(n_u, f),
                  _full_spec(n_u, f), _row_spec(bm, f)],
        out_specs=(_row_spec(bm, 2 * f), _row_spec(bm, f)),
        out_shape=(jax.ShapeDtypeStruct((n_i, 2 * f), bf),
                   jax.ShapeDtypeStruct((n_i, f), jnp.float32)),
    )(VU, u0, u0_share, a_bf, i0)

    spec_u, share_u = pl.pallas_call(
        _s3_body,
        grid=(n_u // bm,),
        in_specs=[_row_spec(bm, n_i), _full_spec(n_i, 2 * f),
                  _row_spec(bm, f), _row_spec(bm, f), _row_spec(bm, f)],
        out_specs=(_row_spec(bm, f), _row_spec(bm, f)),
        out_shape=(jax.ShapeDtypeStruct((n_u, f), jnp.float32),
                   jax.ShapeDtypeStruct((n_u, f), jnp.float32)),
    )(UV, bb_bf, u0, u0_share, a_bf)

    return share_u, spec_u, spec_i


def kernel(source_UV, source_VU, target_UV, target_VU, source_user_emb,
           target_user_emb, source_item_emb, target_item_emb,
           source_user_emb_share, target_user_emb_share):
    s_share_u, s_spec_u, s_spec_i = _domain(
        source_UV, source_VU, source_user_emb, source_user_emb_share,
        source_item_emb)
    t_share_u, t_spec_u, t_spec_i = _domain(
        target_UV, target_VU, target_user_emb, target_user_emb_share,
        target_item_emb)
    return (s_share_u, s_spec_u, s_spec_i, t_share_u, t_spec_u, t_spec_i)


# speedup vs baseline: 1.1461x; 1.1461x over previous
import jax
import jax.numpy as jnp
from jax.experimental import pallas as pl

_BM = 512


def _halves(ref):
    k = ref.shape[0] // 2
    return ref[pl.ds(0, k), :], ref[pl.ds(k, k), :]


def _split_dot(m1, m2, rhs_ref, cast):
    r1, r2 = _halves(rhs_ref)
    if cast:
        r1, r2 = r1.astype(jnp.bfloat16), r2.astype(jnp.bfloat16)
    dims = (((1,), (0,)), ((), ()))
    acc = jax.lax.dot_general(m1, r1, dims,
                              preferred_element_type=jnp.float32)
    acc += jax.lax.dot_general(m2, r2, dims,
                               preferred_element_type=jnp.float32)
    return acc


def _s1_body(uv1_ref, uv2_ref, i0_ref, a_ref):
    uv1 = uv1_ref[...].astype(jnp.bfloat16)
    uv2 = uv2_ref[...].astype(jnp.bfloat16)
    a_ref[...] = _split_dot(uv1, uv2, i0_ref, cast=True).astype(jnp.bfloat16)


def _s2_body(vu1_ref, vu2_ref, u0f_ref, u0sf_ref, ab_ref, i0_ref, bb_ref,
             spec_i_ref):
    vu1 = vu1_ref[...].astype(jnp.bfloat16)
    vu2 = vu2_ref[...].astype(jnp.bfloat16)
    b = _split_dot(vu1, vu2, u0f_ref, cast=True)
    bs = _split_dot(vu1, vu2, u0sf_ref, cast=True)
    d = _split_dot(vu1, vu2, ab_ref, cast=False)
    f = b.shape[1]
    bb_ref[:, :f] = b.astype(jnp.bfloat16)
    bb_ref[:, f:] = bs.astype(jnp.bfloat16)
    spec_i_ref[...] = (i0_ref[...] + b + d) * (1.0 / 3.0)


def _s3_body(uv1_ref, uv2_ref, bb_ref, u0_ref, u0s_ref, ab_ref, spec_u_ref,
             share_u_ref):
    uv1 = uv1_ref[...].astype(jnp.bfloat16)
    uv2 = uv2_ref[...].astype(jnp.bfloat16)
    acc = _split_dot(uv1, uv2, bb_ref, cast=False)
    f = u0_ref.shape[1]
    a = ab_ref[...].astype(jnp.float32)
    spec_u_ref[...] = (u0_ref[...] + a + acc[:, :f]) * (1.0 / 3.0)
    share_u_ref[...] = (u0s_ref[...] + a + acc[:, f:]) * (1.0 / 3.0)


def _row_spec(bm, w):
    return pl.BlockSpec((bm, w), lambda i: (i, 0))


def _half_specs(bm, k):
    return [pl.BlockSpec((bm, k // 2), lambda i: (i, 0)),
            pl.BlockSpec((bm, k // 2), lambda i: (i, 1))]


def _full_spec(k, w):
    return pl.BlockSpec((k, w), lambda i: (0, 0))


def _domain(UV, VU, u0, u0_share, i0):
    n_u, f = u0.shape
    n_i = i0.shape[0]
    bm = _BM
    bf = jnp.bfloat16

    a_bf = pl.pallas_call(
        _s1_body,
        grid=(n_u // bm,),
        in_specs=_half_specs(bm, n_i) + [_full_spec(n_i, f)],
        out_specs=_row_spec(bm, f),
        out_shape=jax.ShapeDtypeStruct((n_u, f), bf),
    )(UV, UV, i0)

    bb_bf, spec_i = pl.pallas_call(
        _s2_body,
        grid=(n_i // bm,),
        in_specs=_half_specs(bm, n_u) + [
            _full_spec(n_u, f), _full_spec(n_u, f), _full_spec(n_u, f),
            _row_spec(bm, f)],
        out_specs=(_row_spec(bm, 2 * f), _row_spec(bm, f)),
        out_shape=(jax.ShapeDtypeStruct((n_i, 2 * f), bf),
                   jax.ShapeDtypeStruct((n_i, f), jnp.float32)),
    )(VU, VU, u0, u0_share, a_bf, i0)

    spec_u, share_u = pl.pallas_call(
        _s3_body,
        grid=(n_u // bm,),
        in_specs=_half_specs(bm, n_i) + [
            _full_spec(n_i, 2 * f),
            _row_spec(bm, f), _row_spec(bm, f), _row_spec(bm, f)],
        out_specs=(_row_spec(bm, f), _row_spec(bm, f)),
        out_shape=(jax.ShapeDtypeStruct((n_u, f), jnp.float32),
                   jax.ShapeDtypeStruct((n_u, f), jnp.float32)),
    )(UV, UV, bb_bf, u0, u0_share, a_bf)

    return share_u, spec_u, spec_i


def kernel(source_UV, source_VU, target_UV, target_VU, source_user_emb,
           target_user_emb, source_item_emb, target_item_emb,
           source_user_emb_share, target_user_emb_share):
    s_share_u, s_spec_u, s_spec_i = _domain(
        source_UV, source_VU, source_user_emb, source_user_emb_share,
        source_item_emb)
    t_share_u, t_spec_u, t_spec_i = _domain(
        target_UV, target_VU, target_user_emb, target_user_emb_share,
        target_item_emb)
    return (s_share_u, s_spec_u, s_spec_i, t_share_u, t_spec_u, t_spec_i)
